# baseline (speedup 1.0000x reference)
import functools

import jax
import jax.numpy as jnp
from jax import lax
from jax.experimental import pallas as pl
from jax.experimental.pallas import tpu as pltpu
from jax.experimental.pallas import tpu_sc as plsc

L = 16
D = 64
W = 128
CHUNK = 128
NG = 4
NO = 2


def _lane_shuffle(x, perm):
    dnums = lax.GatherDimensionNumbers(
        offset_dims=(), collapsed_slice_dims=(0,), start_index_map=(0,))
    return lax.gather(x, perm[:, None], dnums, slice_sizes=(1,),
                      mode=lax.GatherScatterMode.PROMISE_IN_BOUNDS)


def _sc_embed_norm(table2, idx2d, *, n_rows):
    info = plsc.get_sparse_core_info()
    nc, ns = info.num_cores, info.num_subcores
    nw = nc * ns
    per_w = n_rows // nw
    assert per_w % (CHUNK * NG) == 0
    n_chunks = per_w // CHUNK
    idx_rows_per_w = per_w // W

    mesh = plsc.VectorSubcoreMesh(core_axis_name="c", subcore_axis_name="s")

    @functools.partial(
        pl.kernel,
        out_type=jax.ShapeDtypeStruct((n_rows // 2, W), jnp.float32),
        mesh=mesh,
        scratch_types=[
            pltpu.VMEM((idx_rows_per_w, W), jnp.int32),
            pltpu.VMEM((NG, W), jnp.int32),
            pltpu.VMEM((NG, CHUNK, W), jnp.float32),
            pltpu.VMEM((NO, CHUNK // 2, W), jnp.float32),
            pltpu.SemaphoreType.DMA((NG,)),
            pltpu.SemaphoreType.DMA((NO,)),
        ],
        compiler_params=pltpu.CompilerParams(
            needs_layout_passes=False, use_tc_tiling_on_sc=True),
    )
    def k(table_hbm, idx_hbm, out_hbm, idx_v, m_v, rows_v, obuf, gsem, ssem):
        iota = lax.iota(jnp.int32, L)
        perms = [iota ^ sh for sh in (8, 4, 2, 1)]
        wid = lax.axis_index("s") * nc + lax.axis_index("c")
        pltpu.sync_copy(idx_hbm.at[pl.ds(wid * idx_rows_per_w, idx_rows_per_w)],
                        idx_v)
        out_base = wid * (per_w // 2)

        def prep_gather(c, gb):
            for i in range(W // L):
                m_v[gb, pl.ds(i * L, L)] = (
                    idx_v[c, pl.ds(i * L, L)] >> jnp.int32(1))
            pltpu.async_copy(table_hbm.at[m_v.at[gb]], rows_v.at[gb],
                             gsem.at[gb])

        def drain_gather(gb):
            pltpu.make_async_copy(table_hbm.at[m_v.at[gb]], rows_v.at[gb],
                                  gsem.at[gb]).wait()

        def start_store(c, ob):
            pltpu.async_copy(
                obuf.at[ob],
                out_hbm.at[pl.ds(out_base + c * (CHUNK // 2), CHUNK // 2)],
                ssem.at[ob])

        def wait_store(c, ob):
            pltpu.make_async_copy(
                obuf.at[ob],
                out_hbm.at[pl.ds(out_base + c * (CHUNK // 2), CHUNK // 2)],
                ssem.at[ob]).wait()

        def compute(c, gb, ob):
            def block(q, _):
                hv = (idx_v[c, pl.ds(q * L, L)] & jnp.int32(1)) * jnp.int32(D)
                for j in range(L):
                    r = q * L + j
                    hoff = hv[j]
                    v = [rows_v[gb, r, pl.ds(hoff + i * L, L)]
                         for i in range(D // L)]
                    s = v[0] * v[0]
                    for i in range(1, D // L):
                        s = s + v[i] * v[i]
                    for p in perms:
                        s = s + _lane_shuffle(s, p)
                    bits = plsc.bitcast(s, jnp.int32)
                    y = plsc.bitcast(jnp.int32(0x5F3759DF) - (bits >> 1),
                                     jnp.float32)
                    hs = s * jnp.float32(0.5)
                    y = y * (jnp.float32(1.5) - hs * y * y)
                    y = y * (jnp.float32(1.5) - hs * y * y)
                    for i in range(D // L):
                        obuf[ob, q * (L // 2) + j // 2,
                             pl.ds((j % 2) * D + i * L, L)] = v[i] * y
                return ()

            lax.fori_loop(0, CHUNK // L, block, (), unroll=2)

        prep_gather(0, 0)
        prep_gather(1, 1)

        def outer_body(o, _):
            for u in range(NG):
                c = o * NG + u
                gb = u
                ob = u % NO

                @pl.when(c + 2 < n_chunks)
                def _():
                    prep_gather(c + 2, (u + 2) % NG)

                drain_gather(gb)

                @pl.when(c >= NO)
                def _():
                    wait_store(c - NO, ob)

                compute(c, gb, ob)
                start_store(c, ob)
            return ()

        lax.fori_loop(0, n_chunks // NG, outer_body, (), unroll=False)
        wait_store(n_chunks - 2, (n_chunks - 2) % NO)
        wait_store(n_chunks - 1, (n_chunks - 1) % NO)

    return k(table2, idx2d)


def kernel(x, table):
    b, h = x.shape
    n = b * h
    v, d = table.shape
    idx2d = x.reshape(n // W, W).astype(jnp.int32)
    table2 = table.reshape(v // 2, W)
    out = _sc_embed_norm(table2, idx2d, n_rows=n)
    return out.reshape(b, h, d)
